# baseline (speedup 1.0000x reference)
import functools

import jax
import jax.numpy as jnp
from jax import lax
from jax.experimental import pallas as pl
from jax.experimental.pallas import tpu as pltpu
from jax.experimental.pallas import tpu_sc as plsc

_NUM_CORES = 2
_NUM_SUBCORES = 16
_NUM_WORKERS = _NUM_CORES * _NUM_SUBCORES

_CHUNK = 128


def _gather_kernel(n_rows, d, table_hbm, idx_hbm, out_hbm, idx_v, rows_v, sem):
    b_per_w = n_rows // _NUM_WORKERS
    n_chunks = b_per_w // _CHUNK
    wid = lax.axis_index("s") * _NUM_CORES + lax.axis_index("c")
    base = wid * b_per_w

    pltpu.sync_copy(idx_hbm.at[pl.ds(base, b_per_w)], idx_v)

    @pl.loop(0, n_chunks)
    def _chunk_loop(j):
        pltpu.async_copy(
            table_hbm.at[idx_v.at[pl.ds(j * _CHUNK, _CHUNK)]], rows_v, sem
        ).wait()
        pltpu.sync_copy(rows_v, out_hbm.at[pl.ds(base + j * _CHUNK, _CHUNK)])


def _embedding_gather(idx_flat, W):
    n_rows = idx_flat.shape[0]
    d = W.shape[1]
    b_per_w = n_rows // _NUM_WORKERS
    n_chunks = b_per_w // _CHUNK

    mesh = plsc.VectorSubcoreMesh(core_axis_name="c", subcore_axis_name="s")
    kern = pl.kernel(
        functools.partial(_gather_kernel, n_rows, d),
        out_type=jax.ShapeDtypeStruct((n_rows, d), jnp.float32),
        mesh=mesh,
        scratch_types=[
            pltpu.VMEM((b_per_w,), jnp.int32),
            pltpu.VMEM((_CHUNK, d), jnp.float32),
            pltpu.SemaphoreType.DMA,
        ],
        compiler_params=pltpu.CompilerParams(use_tc_tiling_on_sc=False),
    )
    return kern(W, idx_flat)


def kernel(input, W):
    n = input.shape[0] * input.shape[1]
    idx_flat = input.reshape(n).astype(jnp.int32)
    out = _embedding_gather(idx_flat, W)
    return out.reshape(input.shape[0], input.shape[1], W.shape[1])


# speedup vs baseline: 1.0789x; 1.0789x over previous
import functools

import jax
import jax.numpy as jnp
from jax import lax
from jax.experimental import pallas as pl
from jax.experimental.pallas import tpu as pltpu
from jax.experimental.pallas import tpu_sc as plsc

_NUM_CORES = 2
_NUM_SUBCORES = 16
_NUM_WORKERS = _NUM_CORES * _NUM_SUBCORES

_CHUNK = 128


_NBUF = 4


def _gather_kernel(
    n_rows, d, table_hbm, idx_hbm, out_hbm, idx_v, rows_v, gsem, ssem
):
    b_per_w = n_rows // _NUM_WORKERS
    n_chunks = b_per_w // _CHUNK
    wid = lax.axis_index("s") * _NUM_CORES + lax.axis_index("c")
    base = wid * b_per_w

    pltpu.sync_copy(idx_hbm.at[pl.ds(base, b_per_w)], idx_v)

    def _gather(j, b):
        pltpu.async_copy(
            table_hbm.at[idx_v.at[pl.ds(j * _CHUNK, _CHUNK)]],
            rows_v.at[b],
            gsem.at[b],
        )

    def _store(j, b):
        pltpu.async_copy(
            rows_v.at[b], out_hbm.at[pl.ds(base + j * _CHUNK, _CHUNK)], ssem.at[b]
        )

    for b in range(_NBUF):
        _gather(b, b)

    @pl.loop(0, n_chunks - _NBUF, step=_NBUF)
    def _chunk_loop(j0):
        for b in range(_NBUF):
            j = j0 + b
            pltpu.make_async_copy(
                table_hbm.at[idx_v.at[pl.ds(j * _CHUNK, _CHUNK)]],
                rows_v.at[b],
                gsem.at[b],
            ).wait()
            _store(j, b)
            pltpu.make_async_copy(
                rows_v.at[b],
                out_hbm.at[pl.ds(base, _CHUNK)],
                ssem.at[b],
            ).wait()
            _gather(j + _NBUF, b)

    for b in range(_NBUF):
        j = n_chunks - _NBUF + b
        pltpu.make_async_copy(
            table_hbm.at[idx_v.at[pl.ds(j * _CHUNK, _CHUNK)]],
            rows_v.at[b],
            gsem.at[b],
        ).wait()
        _store(j, b)
    for b in range(_NBUF):
        j = n_chunks - _NBUF + b
        pltpu.make_async_copy(
            rows_v.at[b],
            out_hbm.at[pl.ds(base + j * _CHUNK, _CHUNK)],
            ssem.at[b],
        ).wait()


def _embedding_gather(idx_flat, W):
    n_rows = idx_flat.shape[0]
    d = W.shape[1]
    b_per_w = n_rows // _NUM_WORKERS
    n_chunks = b_per_w // _CHUNK

    mesh = plsc.VectorSubcoreMesh(core_axis_name="c", subcore_axis_name="s")
    kern = pl.kernel(
        functools.partial(_gather_kernel, n_rows, d),
        out_type=jax.ShapeDtypeStruct((n_rows, d), jnp.float32),
        mesh=mesh,
        scratch_types=[
            pltpu.VMEM((b_per_w,), jnp.int32),
            pltpu.VMEM((_NBUF, _CHUNK, d), jnp.float32),
            pltpu.SemaphoreType.DMA((_NBUF,)),
            pltpu.SemaphoreType.DMA((_NBUF,)),
        ],
        compiler_params=pltpu.CompilerParams(use_tc_tiling_on_sc=False),
    )
    return kern(W, idx_flat)


def kernel(input, W):
    n = input.shape[0] * input.shape[1]
    idx_flat = input.reshape(n).astype(jnp.int32)
    out = _embedding_gather(idx_flat, W)
    return out.reshape(input.shape[0], input.shape[1], W.shape[1])
